# baseline (speedup 1.0000x reference)
import functools

import jax
import jax.numpy as jnp
from jax import lax
from jax.experimental import pallas as pl
from jax.experimental.pallas import tpu as pltpu
from jax.experimental.pallas import tpu_sc as plsc

_INFO = plsc.get_sparse_core_info()
_NC = _INFO.num_cores
_NS = _INFO.num_subcores
_NW = _NC * _NS


def _make_gather(total, dim, chunk, k_sub):
    assert total % (_NW * chunk) == 0
    assert chunk % k_sub == 0
    sub = chunk // k_sub
    per_w = total // _NW
    n_chunks = per_w // chunk
    mesh = plsc.VectorSubcoreMesh(core_axis_name="c", subcore_axis_name="s")

    @functools.partial(
        pl.kernel,
        mesh=mesh,
        compiler_params=pltpu.CompilerParams(use_tc_tiling_on_sc=False),
        out_type=jax.ShapeDtypeStruct((total, dim), jnp.float32),
        scratch_types=[
            pltpu.VMEM((per_w,), jnp.int32),
            pltpu.VMEM((chunk, dim), jnp.float32),
            pltpu.VMEM((chunk, dim), jnp.float32),
            pltpu.SemaphoreType.DMA,
            pltpu.SemaphoreType.DMA,
            pltpu.SemaphoreType.DMA,
            pltpu.SemaphoreType.DMA,
        ],
    )
    def k(ids_hbm, table_hbm, out_hbm, ids_v, rows0, rows1, g0, g1, s0, s1):
        wid = lax.axis_index("s") * _NC + lax.axis_index("c")
        base = wid * per_w
        rows = (rows0, rows1)
        gsem = (g0, g1)
        ssem = (s0, s1)

        pltpu.sync_copy(ids_hbm.at[pl.ds(base, per_w)], ids_v)

        def issue_gather(g):
            b = g % 2
            return [
                pltpu.async_copy(
                    table_hbm.at[ids_v.at[pl.ds(g * chunk + j * sub, sub)]],
                    rows[b].at[pl.ds(j * sub, sub)], gsem[b])
                for j in range(k_sub)
            ]

        def issue_store(g):
            b = g % 2
            return pltpu.async_copy(
                rows[b], out_hbm.at[pl.ds(base + g * chunk, chunk)],
                ssem[b])

        gh = [None] * n_chunks
        sh = [None] * n_chunks
        gh[0] = issue_gather(0)
        for g in range(n_chunks):
            for h in gh[g]:
                h.wait()
            if g + 1 < n_chunks:
                if g >= 1:
                    sh[g - 1].wait()
                gh[g + 1] = issue_gather(g + 1)
            sh[g] = issue_store(g)
        sh[n_chunks - 2].wait()
        sh[n_chunks - 1].wait()

    return k


def kernel(ids, embeddings):
    batch, hist = ids.shape
    vocab, dim = embeddings.shape
    total = batch * hist
    flat_ids = ids.reshape(total)
    gathered = _make_gather(total, dim, 1600, 8)(flat_ids, embeddings)
    return gathered.reshape(batch, hist, dim)


# speedup vs baseline: 1.7135x; 1.7135x over previous
import functools

import jax
import jax.numpy as jnp
from jax import lax
from jax.experimental import pallas as pl
from jax.experimental.pallas import tpu as pltpu
from jax.experimental.pallas import tpu_sc as plsc

_INFO = plsc.get_sparse_core_info()
_NC = _INFO.num_cores
_NS = _INFO.num_subcores
_NW = _NC * _NS


def _make_gather(batch, hist, dim):
    blk = batch // _NW
    total = batch * hist
    mesh = plsc.VectorSubcoreMesh(core_axis_name="c", subcore_axis_name="s")

    @functools.partial(
        pl.kernel,
        mesh=mesh,
        compiler_params=pltpu.CompilerParams(use_tc_tiling_on_sc=False),
        out_type=jax.ShapeDtypeStruct((total, dim), jnp.float32),
        scratch_types=[
            pltpu.VMEM((hist * blk,), jnp.int32),
            pltpu.VMEM((blk, dim), jnp.float32),
            pltpu.VMEM((blk, dim), jnp.float32),
            pltpu.SemaphoreType.DMA,
            pltpu.SemaphoreType.DMA,
            pltpu.SemaphoreType.DMA,
            pltpu.SemaphoreType.DMA,
            pltpu.SemaphoreType.DMA,
        ],
    )
    def k(ids_hbm, table_hbm, out_hbm, ids_v, rows0, rows1, lsem, g0, g1,
          s0, s1):
        wid = lax.axis_index("s") * _NC + lax.axis_index("c")
        base_b = wid * blk
        rows = (rows0, rows1)
        gsem = (g0, g1)
        ssem = (s0, s1)

        lh = [
            pltpu.async_copy(
                ids_hbm.at[pl.ds(h * batch + base_b, blk)],
                ids_v.at[pl.ds(h * blk, blk)], lsem)
            for h in range(hist)
        ]
        for handle in lh:
            handle.wait()

        def issue_gather(h):
            b = h % 2
            return pltpu.async_copy(
                table_hbm.at[ids_v.at[pl.ds(h * blk, blk)]], rows[b],
                gsem[b])

        def issue_store(h):
            b = h % 2
            return pltpu.async_copy(
                rows[b], out_hbm.at[pl.ds(h * batch + base_b, blk)],
                ssem[b])

        gh = [None] * hist
        sh = [None] * hist
        gh[0] = issue_gather(0)
        for h in range(hist):
            gh[h].wait()
            if h + 1 < hist:
                if h >= 1:
                    sh[h - 1].wait()
                gh[h + 1] = issue_gather(h + 1)
            sh[h] = issue_store(h)
        sh[hist - 2].wait()
        sh[hist - 1].wait()

    return k


def kernel(ids, embeddings):
    batch, hist = ids.shape
    vocab, dim = embeddings.shape
    ids_hm = jnp.swapaxes(ids, 0, 1).reshape(batch * hist)
    flat = _make_gather(batch, hist, dim)(ids_hm, embeddings)
    return jnp.swapaxes(flat.reshape(hist, batch, dim), 0, 1)
